# Initial kernel scaffold; baseline (speedup 1.0000x reference)
import jax
import jax.numpy as jnp
from jax.experimental import pallas as pl


def kernel(embed_weight, neighbor_weight, neighbor_mask, nodes_pocket, neighbor_idx):
    raise NotImplementedError("write your pallas kernel here")


# speedup vs baseline: 2.9485x; 2.9485x over previous
import functools

import jax
import jax.numpy as jnp
from jax import lax
from jax.experimental import pallas as pl
from jax.experimental.pallas import tpu as pltpu
from jax.experimental.pallas import tpu_sc as plsc

_N = 50000
_K = 10
_V = 100000
_D = 128

_NC, _NS = 2, 16
_NW = _NC * _NS
_NPW = 1568
_NPAD = _NW * _NPW
_C = 16
_KP = _K + 1
_NCHUNK = _NPW // _C
_KW = 16

_mesh = plsc.VectorSubcoreMesh(
    core_axis_name="c", subcore_axis_name="s", num_cores=_NC, num_subcores=_NS
)


@functools.partial(
    pl.kernel,
    out_type=jax.ShapeDtypeStruct((_NPAD, _D), jnp.float32),
    mesh=_mesh,
    scratch_types=[
        pltpu.VMEM((_C * _KP,), jnp.int32),
        pltpu.VMEM((_C * _KW,), jnp.float32),
        pltpu.VMEM((_C * _KW,), jnp.float32),
        pltpu.VMEM((_C * _KP, _D), jnp.float32),
        pltpu.VMEM((_C, _D), jnp.float32),
        pltpu.SemaphoreType.DMA,
    ],
)
def _sc_aggregate(idx_hbm, w_hbm, m_hbm, table_hbm, out_hbm,
                  idx_v, w_v, m_v, rows_v, out_v, sem):
    wid = lax.axis_index("s") * _NC + lax.axis_index("c")

    def chunk_body(g, carry):
        node_base = wid * _NPW + g * _C
        pltpu.sync_copy(idx_hbm.at[pl.ds(node_base * _KP, _C * _KP)], idx_v)
        pltpu.sync_copy(w_hbm.at[pl.ds(node_base * _KW, _C * _KW)], w_v)
        pltpu.sync_copy(m_hbm.at[pl.ds(node_base * _KW, _C * _KW)], m_v)
        half = (_C * _KP) // 2
        cp0 = pltpu.async_copy(
            table_hbm.at[idx_v.at[pl.ds(0, half)]],
            rows_v.at[pl.ds(0, half), :], sem)
        cp1 = pltpu.async_copy(
            table_hbm.at[idx_v.at[pl.ds(half, half)]],
            rows_v.at[pl.ds(half, half), :], sem)
        cp0.wait()
        cp1.wait()

        def node_body(i, carry2):
            woff = i * _KW
            wmv = w_v[pl.ds(woff, _KW)] * m_v[pl.ds(woff, _KW)]
            wks = [wmv[k] for k in range(_KP)]
            denom = wks[0]
            for k in range(1, _KP):
                denom = denom + wks[k]
            denom_v = jnp.full((16,), denom, jnp.float32)
            inv = 1.0 / jnp.maximum(denom_v, 1e-6)
            rbase = i * _KP
            acc = [None] * (_D // 16)
            for k in range(_KP):
                wk = wks[k]
                for d in range(_D // 16):
                    seg = rows_v[rbase + k, pl.ds(d * 16, 16)]
                    acc[d] = wk * seg if k == 0 else acc[d] + wk * seg
            for d in range(_D // 16):
                out_v[i, pl.ds(d * 16, 16)] = acc[d] * inv
            return carry2

        lax.fori_loop(0, _C, node_body, 0)
        pltpu.sync_copy(out_v, out_hbm.at[pl.ds(node_base, _C), :])
        return carry

    lax.fori_loop(0, _NCHUNK, chunk_body, 0)


def kernel(embed_weight, neighbor_weight, neighbor_mask, nodes_pocket, neighbor_idx):
    idx_all = jnp.concatenate(
        [neighbor_idx.astype(jnp.int32),
         nodes_pocket.astype(jnp.int32)[:, None]], axis=1)
    idx_all = jnp.pad(idx_all, ((0, _NPAD - _N), (0, 0)))
    ones = jnp.ones((_N, 1), jnp.float32)
    w_all = jnp.pad(jnp.concatenate([neighbor_weight, ones], axis=1),
                    ((0, _NPAD - _N), (0, _KW - _KP)))
    m_all = jnp.pad(jnp.concatenate([neighbor_mask, ones], axis=1),
                    ((0, _NPAD - _N), (0, _KW - _KP)))
    out = _sc_aggregate(idx_all.reshape(-1), w_all.reshape(-1),
                        m_all.reshape(-1), embed_weight)
    return out[:_N]
